# baseline (speedup 1.0000x reference)
import functools

import jax
import jax.numpy as jnp
import numpy as np
from jax import lax
from jax.experimental import pallas as pl
from jax.experimental.pallas import tpu as pltpu
from jax.experimental.pallas import tpu_sc as plsc

N_BINS = 15
INV_TEMP = 0.5
N_ROWS = 1_000_000
N_CLS = 100

N_HALF = N_ROWS // 2

RB = 20000
NB = N_HALF // RB

NC = 2
NS = 16
NW = NC * NS
CHUNK = 15632
LAST_CHUNK = N_HALF - (NW - 1) * CHUNK
NKEY = 32
ACC = NKEY * 16
PART_W = 2 * NKEY

_BOUNDS = np.linspace(0.0, 1.0, N_BINS + 1).astype(np.float32)


def _stage1_body(logits_ref, labels_ref, conf_ref, key_ref):
    xt = logits_ref[...].T
    m = jnp.max(xt, axis=0, keepdims=True)
    labels = labels_ref[0]
    onehot = lax.broadcasted_iota(jnp.int32, xt.shape, 0) == labels
    hit = jnp.where(onehot & (xt == m), 1.0, 0.0)
    c_log2e = jnp.float32(INV_TEMP * 1.4426950408889634)
    ex = jnp.exp2(xt * c_log2e - m * c_log2e)
    ones_v = jnp.ones((1, N_CLS), jnp.float32)
    corr = jnp.dot(ones_v, hit)
    s = jnp.dot(ones_v, ex)
    conf = 1.0 / s
    conf = jnp.where(conf == 1.0, jnp.float32(0.999999), conf)
    b = jnp.minimum(
        jnp.floor(conf * jnp.float32(N_BINS)).astype(jnp.int32), N_BINS - 1)
    conf_ref[0] = conf
    key_ref[0] = b + 16 * corr.astype(jnp.int32)


_stage1 = pl.pallas_call(
    _stage1_body,
    grid=(NB,),
    in_specs=[
        pl.BlockSpec((RB, N_CLS), lambda i: (i, 0)),
        pl.BlockSpec((1, 1, RB), lambda i: (i, 0, 0)),
    ],
    out_specs=[
        pl.BlockSpec((1, 1, RB), lambda i: (i, 0, 0)),
        pl.BlockSpec((1, 1, RB), lambda i: (i, 0, 0)),
    ],
    out_shape=[
        jax.ShapeDtypeStruct((NB, 1, RB), jnp.float32),
        jax.ShapeDtypeStruct((NB, 1, RB), jnp.int32),
    ],
    compiler_params=pltpu.CompilerParams(
        dimension_semantics=("parallel",)),
)


def _hist_body(conf_hbm, key_hbm, out_hbm, conf_v, key_v, acc_c, acc_f, part_v):
    c = lax.axis_index("c")
    s = lax.axis_index("s")
    wid = s * NC + c
    base = wid * CHUNK

    zero = jnp.zeros((16,), jnp.float32)
    for r in range(NKEY):
        acc_c[pl.ds(r * 16, 16)] = zero
        acc_f[pl.ds(r * 16, 16)] = zero

    lane = lax.iota(jnp.int32, 16)
    ones = jnp.full((16,), 1.0, jnp.float32)

    def body(j, carry):
        off = j * 16
        kv = key_v[pl.ds(off, 16)]
        cf = conf_v[pl.ds(off, 16)]
        idx = kv * 16 + lane
        plsc.addupdate_scatter(acc_c, [idx], ones)
        plsc.addupdate_scatter(acc_f, [idx], cf)
        return carry

    @pl.when(wid < NW - 1)
    def _full():
        pltpu.sync_copy(conf_hbm.at[pl.ds(base, CHUNK)], conf_v)
        pltpu.sync_copy(key_hbm.at[pl.ds(base, CHUNK)], key_v)
        lax.fori_loop(0, CHUNK // 16, body, 0)

    @pl.when(wid == NW - 1)
    def _tail():
        pltpu.sync_copy(conf_hbm.at[pl.ds(base, LAST_CHUNK)],
                        conf_v.at[pl.ds(0, LAST_CHUNK)])
        pltpu.sync_copy(key_hbm.at[pl.ds(base, LAST_CHUNK)],
                        key_v.at[pl.ds(0, LAST_CHUNK)])
        lax.fori_loop(0, LAST_CHUNK // 16, body, 0)

    for h in range(2):
        tot_c = zero
        tot_f = zero
        for l in range(16):
            gi = (h * 16 + lane) * 16 + l
            tot_c = tot_c + plsc.load_gather(acc_c, [gi])
            tot_f = tot_f + plsc.load_gather(acc_f, [gi])
        part_v[pl.ds(h * 16, 16)] = tot_c
        part_v[pl.ds(NKEY + h * 16, 16)] = tot_f
    pltpu.sync_copy(part_v, out_hbm.at[pl.ds(wid * PART_W, PART_W)])


@functools.cache
def _get_hist():
    return pl.kernel(
        _hist_body,
        out_type=jax.ShapeDtypeStruct((NW * PART_W,), jnp.float32),
        mesh=plsc.VectorSubcoreMesh(core_axis_name="c", subcore_axis_name="s"),
        compiler_params=pltpu.CompilerParams(needs_layout_passes=False),
        scratch_types=[
            pltpu.VMEM((CHUNK,), jnp.float32),
            pltpu.VMEM((CHUNK,), jnp.int32),
            pltpu.VMEM((ACC,), jnp.float32),
            pltpu.VMEM((ACC,), jnp.float32),
            pltpu.VMEM((PART_W,), jnp.float32),
        ],
    )


def _combine_body(p_ref, out_ref):
    p = p_ref[...]
    row = lax.broadcasted_iota(jnp.int32, p.shape, 0) % 4
    cnt_lo = jnp.sum(jnp.where(row == 0, p, 0.0), axis=0)
    cnt_hi = jnp.sum(jnp.where(row == 1, p, 0.0), axis=0)
    cf_lo = jnp.sum(jnp.where(row == 2, p, 0.0), axis=0)
    cf_hi = jnp.sum(jnp.where(row == 3, p, 0.0), axis=0)
    cnt = cnt_lo + cnt_hi
    cf = cf_lo + cf_hi
    cr = cnt_hi
    safe = jnp.maximum(cnt, 1.0)
    acc = jnp.clip(cr / safe, 0.01, 0.99)
    avgc = cf / safe
    prop = cnt / jnp.float32(N_ROWS)
    contrib = jnp.where(cnt > 10.0, jnp.abs(avgc - acc) * prop, 0.0)
    lanei = lax.broadcasted_iota(jnp.int32, (16,), 0)
    contrib = jnp.where(lanei < N_BINS, contrib, 0.0)
    out_ref[...] = jnp.sum(contrib.reshape(1, 16), axis=1, keepdims=True)


_combine = pl.pallas_call(
    _combine_body,
    in_specs=[pl.BlockSpec((8 * NW, 16), lambda: (0, 0))],
    out_specs=pl.BlockSpec((1, 1), lambda: (0, 0)),
    out_shape=jax.ShapeDtypeStruct((1, 1), jnp.float32),
)


@jax.jit
def kernel(logits, labels):
    hist = _get_hist()
    lo_logits = lax.slice(logits, (0, 0), (N_HALF, N_CLS))
    hi_logits = lax.slice(logits, (N_HALF, 0), (N_ROWS, N_CLS))
    lo_labels = lax.slice(labels, (0,), (N_HALF,)).reshape(NB, 1, RB)
    hi_labels = lax.slice(labels, (N_HALF,), (N_ROWS,)).reshape(NB, 1, RB)
    conf1, keys1 = _stage1(lo_logits, lo_labels)
    parts1 = hist(conf1.reshape(N_HALF), keys1.reshape(N_HALF))
    conf2, keys2 = _stage1(hi_logits, hi_labels)
    parts2 = hist(conf2.reshape(N_HALF), keys2.reshape(N_HALF))
    parts = jnp.concatenate([parts1, parts2])
    ece = _combine(parts.reshape(8 * NW, 16))
    return ece.reshape(1)


# speedup vs baseline: 1.3477x; 1.3477x over previous
import functools

import jax
import jax.numpy as jnp
import numpy as np
from jax import lax
from jax.experimental import pallas as pl
from jax.experimental.pallas import tpu as pltpu
from jax.experimental.pallas import tpu_sc as plsc

N_BINS = 15
INV_TEMP = 0.5
N_ROWS = 1_000_000
N_CLS = 100

RB = 20000
NB = N_ROWS // RB

NC = 2
NS = 16
NW = NC * NS
CHUNK = 32000
LAST_CHUNK = N_ROWS - (NW - 1) * CHUNK
NKEY = 32
ACC = NKEY * 16
PART_W = 2 * NKEY

_BOUNDS = np.linspace(0.0, 1.0, N_BINS + 1).astype(np.float32)


def _stage1_body(logits_ref, labels_ref, val_ref):
    xt = logits_ref[...].T
    m = jnp.max(xt, axis=0, keepdims=True)
    labels = labels_ref[0]
    onehot = lax.broadcasted_iota(jnp.int32, xt.shape, 0) == labels
    hit = jnp.where(onehot & (xt == m), 1.0, 0.0)
    c_log2e = jnp.float32(INV_TEMP * 1.4426950408889634)
    ex = jnp.exp2(xt * c_log2e - m * c_log2e)
    ones_v = jnp.ones((1, N_CLS), jnp.float32)
    corr = jnp.dot(ones_v, hit)
    s = jnp.dot(ones_v, ex)
    conf = 1.0 / s
    conf = jnp.where(conf == 1.0, jnp.float32(0.999999), conf)
    b = jnp.minimum(
        jnp.floor(conf * jnp.float32(N_BINS)).astype(jnp.int32), N_BINS - 1)
    key = b + 16 * corr.astype(jnp.int32)
    conf_packed = jnp.minimum(conf, jnp.float32(0.999998))
    val_ref[0] = key.astype(jnp.float32) + conf_packed


_stage1 = pl.pallas_call(
    _stage1_body,
    grid=(NB,),
    in_specs=[
        pl.BlockSpec((RB, N_CLS), lambda i: (i, 0)),
        pl.BlockSpec((1, 1, RB), lambda i: (i, 0, 0)),
    ],
    out_specs=pl.BlockSpec((1, 1, RB), lambda i: (i, 0, 0)),
    out_shape=jax.ShapeDtypeStruct((NB, 1, RB), jnp.float32),
    compiler_params=pltpu.CompilerParams(
        dimension_semantics=("parallel",)),
)


def _hist_body(val_hbm, out_hbm, val_v, acc_c, acc_f, part_v):
    c = lax.axis_index("c")
    s = lax.axis_index("s")
    wid = s * NC + c
    base = wid * CHUNK

    zero = jnp.zeros((16,), jnp.float32)
    for r in range(NKEY):
        acc_c[pl.ds(r * 16, 16)] = zero
        acc_f[pl.ds(r * 16, 16)] = zero

    lane = lax.iota(jnp.int32, 16)
    ones = jnp.full((16,), 1.0, jnp.float32)

    def body(j, carry):
        off = j * 16
        vv = val_v[pl.ds(off, 16)]
        kv = vv.astype(jnp.int32)
        cf = vv - kv.astype(jnp.float32)
        idx = kv * 16 + lane
        plsc.addupdate_scatter(acc_c, [idx], ones)
        plsc.addupdate_scatter(acc_f, [idx], cf)
        return carry

    @pl.when(wid < NW - 1)
    def _full():
        pltpu.sync_copy(val_hbm.at[pl.ds(base, CHUNK)], val_v)
        lax.fori_loop(0, CHUNK // 16, body, 0)

    @pl.when(wid == NW - 1)
    def _tail():
        pltpu.sync_copy(val_hbm.at[pl.ds(base, LAST_CHUNK)],
                        val_v.at[pl.ds(0, LAST_CHUNK)])
        lax.fori_loop(0, LAST_CHUNK // 16, body, 0)

    for h in range(2):
        tot_c = zero
        tot_f = zero
        for l in range(16):
            gi = (h * 16 + lane) * 16 + l
            tot_c = tot_c + plsc.load_gather(acc_c, [gi])
            tot_f = tot_f + plsc.load_gather(acc_f, [gi])
        part_v[pl.ds(h * 16, 16)] = tot_c
        part_v[pl.ds(NKEY + h * 16, 16)] = tot_f
    pltpu.sync_copy(part_v, out_hbm.at[pl.ds(wid * PART_W, PART_W)])


@functools.cache
def _get_hist():
    return pl.kernel(
        _hist_body,
        out_type=jax.ShapeDtypeStruct((NW * PART_W,), jnp.float32),
        mesh=plsc.VectorSubcoreMesh(core_axis_name="c", subcore_axis_name="s"),
        compiler_params=pltpu.CompilerParams(needs_layout_passes=False),
        scratch_types=[
            pltpu.VMEM((CHUNK,), jnp.float32),
            pltpu.VMEM((ACC,), jnp.float32),
            pltpu.VMEM((ACC,), jnp.float32),
            pltpu.VMEM((PART_W,), jnp.float32),
        ],
    )


def _combine_body(p_ref, out_ref):
    p = p_ref[...]
    row = lax.broadcasted_iota(jnp.int32, p.shape, 0) % 4
    cnt_lo = jnp.sum(jnp.where(row == 0, p, 0.0), axis=0)
    cnt_hi = jnp.sum(jnp.where(row == 1, p, 0.0), axis=0)
    cf_lo = jnp.sum(jnp.where(row == 2, p, 0.0), axis=0)
    cf_hi = jnp.sum(jnp.where(row == 3, p, 0.0), axis=0)
    cnt = cnt_lo + cnt_hi
    cf = cf_lo + cf_hi
    cr = cnt_hi
    safe = jnp.maximum(cnt, 1.0)
    acc = jnp.clip(cr / safe, 0.01, 0.99)
    avgc = cf / safe
    prop = cnt / jnp.float32(N_ROWS)
    contrib = jnp.where(cnt > 10.0, jnp.abs(avgc - acc) * prop, 0.0)
    lanei = lax.broadcasted_iota(jnp.int32, (16,), 0)
    contrib = jnp.where(lanei < N_BINS, contrib, 0.0)
    out_ref[...] = jnp.sum(contrib.reshape(1, 16), axis=1, keepdims=True)


_combine = pl.pallas_call(
    _combine_body,
    in_specs=[pl.BlockSpec((4 * NW, 16), lambda: (0, 0))],
    out_specs=pl.BlockSpec((1, 1), lambda: (0, 0)),
    out_shape=jax.ShapeDtypeStruct((1, 1), jnp.float32),
)


@jax.jit
def kernel(logits, labels):
    labels3 = labels.reshape(NB, 1, RB)
    vals = _stage1(logits, labels3)
    parts = _get_hist()(vals.reshape(N_ROWS))
    ece = _combine(parts.reshape(4 * NW, 16))
    return ece.reshape(1)
